# Initial kernel scaffold; baseline (speedup 1.0000x reference)
import jax
import jax.numpy as jnp
from jax.experimental import pallas as pl


def kernel(X, hash_table, W1, b1, W2, b2, W3, b3, W4, b4):
    raise NotImplementedError("write your pallas kernel here")


# speedup vs baseline: 115.8502x; 115.8502x over previous
import numpy as np
import jax
import jax.numpy as jnp
from jax.experimental import pallas as pl

L = 16
N_MIN = 16
N_MAX = 64
B = 16384
BLK = 2048

_growth = np.exp((np.log(N_MAX) - np.log(N_MIN)) / (L - 1))
_NV = np.floor(np.float32(N_MIN * _growth ** np.arange(L))).astype(np.int64)
_N_ROW = np.repeat(_NV.astype(np.float32), 2).reshape(1, 2 * L)


def _mlp_encode_kernel(x_ref, n_ref, c_ref, w1_ref, b1_ref, w2_ref, b2_ref,
                       w3_ref, b3_ref, w4_ref, b4_ref, o_ref):
    n_row = n_ref[:, :]
    x0 = x_ref[:, 0:1]
    x1 = x_ref[:, 1:2]

    sx = x0 * n_row
    sy = x1 * n_row
    isx = jnp.floor(sx)
    isy = jnp.floor(sy)
    fx = sx - isx
    fy = sy - isy
    px = isx - 2.0 * jnp.floor(isx * 0.5)
    py = isy - 2.0 * jnp.floor(isy * 0.5)
    pxy = px + py - 2.0 * px * py

    cx = 1.0 - fx
    cy = 1.0 - fy
    w0 = cx * cy
    w1 = cx * fy
    w2 = fx * cy
    w3 = fx * fy

    a0 = c_ref[0:1, :]
    a1 = c_ref[1:2, :]
    a2 = c_ref[2:3, :]
    a3 = c_ref[3:4, :]
    d1 = c_ref[5:6, :] - a1
    d2 = c_ref[6:7, :] - a2
    d3 = c_ref[7:8, :] - a3

    h = (w0 * a0 + w1 * a1 + w2 * a2 + w3 * a3
         + (w1 * py) * d1 + (w2 * px) * d2 + (w3 * pxy) * d3)

    def lrelu(v):
        return jnp.where(v >= 0, v, 0.01 * v)

    h = lrelu(jnp.dot(h, w1_ref[:, :], preferred_element_type=jnp.float32)
              + b1_ref[:, :])
    h = lrelu(jnp.dot(h, w2_ref[:, :], preferred_element_type=jnp.float32)
              + b2_ref[:, :])
    h = lrelu(jnp.dot(h, w3_ref[:, :], preferred_element_type=jnp.float32)
              + b3_ref[:, :])
    o = jnp.dot(h, w4_ref[:, :], preferred_element_type=jnp.float32) \
        + b4_ref[:, :]
    o_ref[:, :] = jnp.maximum(o, 0.0)


def kernel(X, hash_table, W1, b1, W2, b2, W3, b3, W4, b4):
    t0 = hash_table[0, :4, :]
    t1 = hash_table[1, :4, :]
    c0 = jnp.tile(t0.reshape(4, 1, 2), (1, L, 1)).reshape(4, 2 * L)
    c1 = jnp.tile(t1.reshape(4, 1, 2), (1, L, 1)).reshape(4, 2 * L)
    C = jnp.concatenate([c0, c1], axis=0)

    grid = (B // BLK,)
    _z = np.int32(0)
    full = lambda shape: pl.BlockSpec(shape, lambda i: (_z, _z))
    out = pl.pallas_call(
        _mlp_encode_kernel,
        grid=grid,
        in_specs=[
            pl.BlockSpec((BLK, 2), lambda i: (i, _z)),
            full((1, 2 * L)),
            full((8, 2 * L)),
            full((32, 64)), full((1, 64)),
            full((64, 64)), full((1, 64)),
            full((64, 64)), full((1, 64)),
            full((64, 3)), full((1, 3)),
        ],
        out_specs=pl.BlockSpec((BLK, 3), lambda i: (i, _z)),
        out_shape=jax.ShapeDtypeStruct((B, 3), jnp.float32),
    )(X, jnp.asarray(_N_ROW), C, W1.T, b1.reshape(1, 64), W2.T, b2.reshape(1, 64),
      W3.T, b3.reshape(1, 64), W4.T, b4.reshape(1, 3))
    return out
